# baseline (speedup 1.0000x reference)
import jax
import jax.numpy as jnp
from jax import lax
from jax.experimental import pallas as pl
from jax.experimental.pallas import tpu as pltpu
from jax.experimental.pallas import tpu_sc as plsc

NC = 2
NS = 16
L = 16
NW = NC * NS

B = 16384
D = 64
BPW = B // NW
C = 128
NCHUNK = BPW // C
NSLICE = D // L


def _body(s_hbm, o_hbm, r_hbm, t_hbm, e_hbm, rw_hbm, tw_hbm, out_hbm, *scratch):
    idx_bufs = [scratch[p * 8: p * 8 + 4] for p in range(2)]
    row_bufs = [scratch[p * 8 + 4: p * 8 + 8] for p in range(2)]
    out_v = scratch[16]
    sems = scratch[17:19]

    cid = lax.axis_index("c")
    sid = lax.axis_index("s")
    wid = sid * NC + cid
    base = wid * BPW

    lane = lax.iota(jnp.int32, L)

    def stage(chunk, p):
        off = base + chunk * C
        for idx_ref, src in zip(idx_bufs[p], (s_hbm, o_hbm, r_hbm, t_hbm)):
            pltpu.sync_copy(src.at[pl.ds(off, C)], idx_ref)
        tables = (e_hbm, e_hbm, rw_hbm, tw_hbm)
        return [pltpu.async_copy(tab.at[idx_ref], rows_ref, sems[p])
                for tab, idx_ref, rows_ref in zip(tables, idx_bufs[p], row_bufs[p])]

    def compute(chunk, p):
        sr, orr, rr, tr = row_bufs[p]

        def blk_body(blk, carry):
            rows = blk * L + lane
            acc = jnp.zeros((L,), jnp.float32)
            for d in range(D):
                dd = jnp.full((L,), d, jnp.int32)
                sv = plsc.load_gather(sr, [rows, dd])
                rv = plsc.load_gather(rr, [rows, dd])
                tv = plsc.load_gather(tr, [rows, dd])
                ov = plsc.load_gather(orr, [rows, dd])
                acc = acc + jnp.abs(sv + rv + tv - ov)
            out_v[pl.ds(chunk * C + blk * L, L)] = acc
            return carry

        lax.fori_loop(0, C // L, blk_body, 0)

    copies = stage(0, 0)
    for c in range(NCHUNK):
        nxt = stage(c + 1, (c + 1) % 2) if c + 1 < NCHUNK else None
        for cp in copies:
            cp.wait()
        compute(c, c % 2)
        copies = nxt

    pltpu.sync_copy(out_v, out_hbm.at[pl.ds(base, BPW)])


def kernel(s, o, r, t, e_weight, r_weight, t_weight):
    scratch = []
    for _ in range(2):
        scratch += [pltpu.VMEM((C,), jnp.int32) for _ in range(4)]
        scratch += [pltpu.VMEM((C, D), jnp.float32) for _ in range(4)]
    scratch += [pltpu.VMEM((BPW,), jnp.float32)]
    scratch += [pltpu.SemaphoreType.DMA, pltpu.SemaphoreType.DMA]

    sc_call = pl.kernel(
        _body,
        out_type=jax.ShapeDtypeStruct((B,), jnp.float32),
        mesh=plsc.VectorSubcoreMesh(core_axis_name="c", subcore_axis_name="s"),
        scratch_types=scratch,
        compiler_params=pltpu.CompilerParams(
            needs_layout_passes=False, use_tc_tiling_on_sc=False),
    )
    return sc_call(s, o, r, t, e_weight, r_weight, t_weight)


# speedup vs baseline: 1.1067x; 1.1067x over previous
import jax
import jax.numpy as jnp
from jax import lax
from jax.experimental import pallas as pl
from jax.experimental.pallas import tpu as pltpu
from jax.experimental.pallas import tpu_sc as plsc

NC = 2
NS = 16
L = 16
NW = NC * NS

B = 16384
D = 64
BPW = B // NW
C = 128
NCHUNK = BPW // C
NSLICE = D // L


def _body(s_hbm, o_hbm, r_hbm, t_hbm, e_hbm, rw_hbm, tw_hbm, out_hbm, *scratch):
    idx = scratch[0:4]
    rows = [scratch[4 + 4 * p: 8 + 4 * p] for p in range(2)]
    tp = scratch[12]
    out_v = scratch[13]
    sems = scratch[14:16]

    cid = lax.axis_index("c")
    sid = lax.axis_index("s")
    wid = sid * NC + cid
    base = wid * BPW

    lane = lax.iota(jnp.int32, L)
    tabs = (e_hbm, e_hbm, rw_hbm, tw_hbm)

    for buf, src in zip(idx, (s_hbm, o_hbm, r_hbm, t_hbm)):
        pltpu.sync_copy(src.at[pl.ds(base, BPW)], buf)

    def stage(c, p):
        for k in range(4):
            pltpu.async_copy(
                tabs[k].at[idx[k].at[pl.ds(c * C, C)]], rows[p][k], sems[p])

    def drain(p):
        for k in range(4):
            pltpu.make_async_copy(
                tabs[k].at[idx[k].at[pl.ds(0, C)]], rows[p][k], sems[p]).wait()

    def compute(c, p):
        sr, orr, rr, tr = rows[p]

        def blk_body(blk, carry):
            for j in range(L):
                row = blk * L + j
                part = None
                for q in range(NSLICE):
                    sl = pl.ds(q * L, L)
                    d = sr[row, sl] + rr[row, sl] + tr[row, sl] - orr[row, sl]
                    a = jnp.abs(d)
                    part = a if part is None else part + a
                plsc.store_scatter(tp, [lane, jnp.full((L,), j, jnp.int32)], part)
            acc = tp[0, :]
            for l in range(1, L):
                acc = acc + tp[l, :]
            out_v[pl.ds(c * C + blk * L, L)] = acc
            return carry

        lax.fori_loop(0, C // L, blk_body, 0)

    stage(0, 0)
    for c in range(NCHUNK):
        if c + 1 < NCHUNK:
            stage(c + 1, (c + 1) % 2)
        drain(c % 2)
        compute(c, c % 2)

    pltpu.sync_copy(out_v, out_hbm.at[pl.ds(base, BPW)])


def kernel(s, o, r, t, e_weight, r_weight, t_weight):
    scratch = [pltpu.VMEM((BPW,), jnp.int32) for _ in range(4)]
    for _ in range(2):
        scratch += [pltpu.VMEM((C, D), jnp.float32) for _ in range(4)]
    scratch += [pltpu.VMEM((L, L), jnp.float32)]
    scratch += [pltpu.VMEM((BPW,), jnp.float32)]
    scratch += [pltpu.SemaphoreType.DMA, pltpu.SemaphoreType.DMA]

    sc_call = pl.kernel(
        _body,
        out_type=jax.ShapeDtypeStruct((B,), jnp.float32),
        mesh=plsc.VectorSubcoreMesh(core_axis_name="c", subcore_axis_name="s"),
        scratch_types=scratch,
        compiler_params=pltpu.CompilerParams(
            needs_layout_passes=False, use_tc_tiling_on_sc=False),
    )
    return sc_call(s, o, r, t, e_weight, r_weight, t_weight)
